# baseline (speedup 1.0000x reference)
import functools

import jax
import jax.numpy as jnp
from jax import lax
from jax.experimental import pallas as pl
from jax.experimental.pallas import tpu as pltpu
from jax.experimental.pallas import tpu_sc as plsc

N_CLS = 1000
D0, D1 = 1024, 26
NC, NS, L = 2, 16, 16
JPC = D1 // NC
WR = 56
NWIN = -(-N_CLS // WR)
SUB = 8
NGRP = D0 // L

_mesh = plsc.VectorSubcoreMesh(
    core_axis_name="c", subcore_axis_name="s", num_cores=NC, num_subcores=NS
)


@functools.partial(
    pl.kernel,
    out_type=jax.ShapeDtypeStruct((D1 * N_CLS, D0), jnp.int32),
    mesh=_mesh,
    compiler_params=pltpu.CompilerParams(needs_layout_passes=False),
    scratch_types=[
        pltpu.VMEM((D0,), jnp.int32),
        pltpu.VMEM((WR, D0), jnp.int32),
        pltpu.VMEM((WR, D0), jnp.int32),
        pltpu.VMEM((D0,), jnp.int32),
        pltpu.VMEM((D0,), jnp.int32),
        pltpu.SemaphoreType.DMA,
        pltpu.SemaphoreType.DMA,
    ],
)
def _onehot_sc(xt_hbm, zeros_hbm, out_hbm, xcol, win0, win1, pos0, pos1,
               sem0, sem1):
    sid = lax.axis_index("s")
    cid = lax.axis_index("c")

    wins = (win0, win1)
    poss = (pos0, pos1)
    sems = (sem0, sem1)
    lanes = lax.iota(jnp.int32, L)
    ones = jnp.ones((L,), jnp.int32)
    zeros = jnp.zeros((L,), jnp.int32)

    def assemble(b, k0, hi):
        win, pos = wins[b], poss[b]

        def body(gi, _):
            xv = xcol[pl.ds(gi * L, L)]
            t = xv - k0
            mask = (t >= 0) & (t < hi)
            plsc.store_scatter(win, [t, gi * L + lanes], ones, mask=mask)
            pos[pl.ds(gi * L, L)] = jnp.where(mask, t, 0)
            return 0

        lax.fori_loop(0, NGRP, body, 0)

    def restore(b):
        win, pos = wins[b], poss[b]

        def body(gi, _):
            t = pos[pl.ds(gi * L, L)]
            plsc.store_scatter(win, [t, gi * L + lanes], zeros)
            return 0

        lax.fori_loop(0, NGRP, body, 0)

    @pl.when(sid < JPC)
    def _():
        j = cid * JPC + sid
        pltpu.sync_copy(xt_hbm.at[pl.ds(j * D0, D0)], xcol)
        pltpu.sync_copy(zeros_hbm, win0)
        pltpu.sync_copy(zeros_hbm, win1)
        descs = [[], []]
        for w in range(NWIN):
            b = w & 1
            for d in descs[b]:
                d.wait()
            if w >= 2:
                restore(b)
            k0 = w * WR
            hi = min(WR, N_CLS - k0)
            assemble(b, k0, hi)
            descs[b] = [
                pltpu.async_copy(
                    wins[b].at[pl.ds(s * SUB, SUB), :],
                    out_hbm.at[pl.ds(j * N_CLS + k0 + s * SUB, SUB)],
                    sems[b],
                )
                for s in range(hi // SUB)
            ]
        for b in (0, 1):
            for d in descs[b]:
                d.wait()


def kernel(x):
    xt = x.astype(jnp.int32).T.reshape(-1)
    z = jnp.zeros((WR, D0), jnp.int32)
    out2 = _onehot_sc(xt, z)
    out3 = out2.reshape(D1, N_CLS, D0)
    return jnp.transpose(out3, (2, 0, 1))


# speedup vs baseline: 1.0239x; 1.0239x over previous
import functools

import jax
import jax.numpy as jnp
from jax import lax
from jax.experimental import pallas as pl
from jax.experimental.pallas import tpu as pltpu
from jax.experimental.pallas import tpu_sc as plsc

N_CLS = 1000
D0, D1 = 1024, 26
NC, NS, L = 2, 16, 16
JPC = D1 // NC
WR = 48
NWIN = -(-N_CLS // WR)
SUB = 8
NGRP = D0 // L

_mesh = plsc.VectorSubcoreMesh(
    core_axis_name="c", subcore_axis_name="s", num_cores=NC, num_subcores=NS
)


@functools.partial(
    pl.kernel,
    out_type=jax.ShapeDtypeStruct((D1 * N_CLS, D0), jnp.int32),
    mesh=_mesh,
    compiler_params=pltpu.CompilerParams(needs_layout_passes=False),
    scratch_types=[
        pltpu.VMEM((D0,), jnp.int32),
        pltpu.VMEM((WR, D0), jnp.int32),
        pltpu.VMEM((WR, D0), jnp.int32),
        pltpu.VMEM((D0,), jnp.int32),
        pltpu.VMEM((D0,), jnp.int32),
        pltpu.SemaphoreType.DMA,
        pltpu.SemaphoreType.DMA,
    ],
)
def _onehot_sc(xt_hbm, zeros_hbm, out_hbm, xcol, win0, win1, pos0, pos1,
               sem0, sem1):
    sid = lax.axis_index("s")
    cid = lax.axis_index("c")

    wins = (win0, win1)
    poss = (pos0, pos1)
    sems = (sem0, sem1)
    lanes = lax.iota(jnp.int32, L)
    ones = jnp.ones((L,), jnp.int32)
    zeros = jnp.zeros((L,), jnp.int32)

    def assemble(b, k0, hi):
        win, pos = wins[b], poss[b]

        def body(gi, _):
            xv = xcol[pl.ds(gi * L, L)]
            t = xv - k0
            mask = (t >= 0) & (t < hi)
            plsc.store_scatter(win, [t, gi * L + lanes], ones, mask=mask)
            pos[pl.ds(gi * L, L)] = jnp.where(mask, t, 0)
            return 0

        lax.fori_loop(0, NGRP, body, 0)

    def restore(b):
        win, pos = wins[b], poss[b]

        def body(gi, _):
            t = pos[pl.ds(gi * L, L)]
            plsc.store_scatter(win, [t, gi * L + lanes], zeros)
            return 0

        lax.fori_loop(0, NGRP, body, 0)

    @pl.when(sid < JPC)
    def _():
        j = cid * JPC + sid
        pltpu.sync_copy(xt_hbm.at[pl.ds(j * D0, D0)], xcol)
        pltpu.sync_copy(zeros_hbm, win0)
        pltpu.sync_copy(zeros_hbm, win1)
        descs = [[], []]
        for w in range(NWIN):
            b = w & 1
            for d in descs[b]:
                d.wait()
            if w >= 2:
                restore(b)
            k0 = w * WR
            hi = min(WR, N_CLS - k0)
            assemble(b, k0, hi)
            descs[b] = [
                pltpu.async_copy(
                    wins[b].at[pl.ds(s * SUB, SUB), :],
                    out_hbm.at[pl.ds(j * N_CLS + k0 + s * SUB, SUB)],
                    sems[b],
                )
                for s in range(hi // SUB)
            ]
        for b in (0, 1):
            for d in descs[b]:
                d.wait()


def kernel(x):
    xt = x.astype(jnp.int32).T.reshape(-1)
    z = jnp.zeros((WR, D0), jnp.int32)
    out2 = _onehot_sc(xt, z)
    out3 = out2.reshape(D1, N_CLS, D0)
    return jnp.transpose(out3, (2, 0, 1))
